# baseline (speedup 1.0000x reference)
import functools

import jax
import jax.numpy as jnp
from jax import lax
from jax.experimental import pallas as pl
from jax.experimental.pallas import tpu as pltpu
from jax.experimental.pallas import tpu_sc as plsc

B, S, D = 4096, 200, 256
NTYPE, NIDX = 5, 50
NIDXP = 56
NTAB = NTYPE * NIDXP * S
DP = D // 2
NC, NS = 2, 16
NW = NC * NS
NB = B // NW
LANES = 16
G0, G1 = 104, 96
MSTR = 208
PAGE_LO = NIDXP * S
PAGE_HI = (NIDXP + NIDX) * S


def _style_mask_body(types_ref, inds_ref, sv_ref, w1_ref, b1_ref, w2_ref,
                     b2_ref, styled_ref, mask_ref, gidx_ref):
    h = jnp.dot(sv_ref[...], w1_ref[...], preferred_element_type=jnp.float32)
    h = jnp.maximum(h + b1_ref[...][None, :], 0.0)
    styled = jnp.dot(h, w2_ref[...], preferred_element_type=jnp.float32)
    styled_ref[...] = styled + b2_ref[...][None, :]
    mask_ref[...] = types_ref[...] == 0
    s_iota = lax.broadcasted_iota(jnp.int32, (B, S), 1)
    gidx_ref[...] = (types_ref[...] * NIDXP + inds_ref[...]) * S + s_iota


def _tc_pre(types, inds, style_vector, w1, b1, w2, b2):
    return pl.pallas_call(
        _style_mask_body,
        out_shape=[
            jax.ShapeDtypeStruct((B, D), jnp.float32),
            jax.ShapeDtypeStruct((B, S), jnp.bool_),
            jax.ShapeDtypeStruct((B, S), jnp.int32),
        ],
    )(types, inds, style_vector, w1, b1, w2, b2)


def _bigtab_body(temb_ref, iemb_ref, pemb_ref, out_ref):
    g = pl.program_id(0)
    trow = jnp.zeros((1, D), jnp.float32)
    for t in range(NTYPE):
        trow = trow + temb_ref[pl.ds(t, 1), :] * jnp.where(g == t, 1.0, 0.0)
    big = (iemb_ref[...][:, None, :] + pemb_ref[...][None, :, :]
           + trow[None, :, :])
    x = big.reshape(8 * S, 8, 32)
    lo = lax.bitcast_convert_type(
        x[:, :, :LANES].astype(jnp.bfloat16), jnp.uint16).astype(jnp.uint32)
    hi = lax.bitcast_convert_type(
        x[:, :, LANES:].astype(jnp.bfloat16), jnp.uint16).astype(jnp.uint32)
    packed = lax.bitcast_convert_type(lo | (hi << 16), jnp.int32)
    out_ref[...] = packed.reshape(8 * S, DP)


def _tc_bigtab(temb, iemb, pemb):
    return pl.pallas_call(
        _bigtab_body,
        grid=(NTYPE, NIDXP // 8),
        in_specs=[
            pl.BlockSpec((NTYPE, D), lambda gt, gi: (0, 0)),
            pl.BlockSpec((8, D), lambda gt, gi: (gi, 0)),
            pl.BlockSpec((S, D), lambda gt, gi: (0, 0)),
        ],
        out_specs=pl.BlockSpec(
            (8 * S, DP), lambda gt, gi: (gt * (NIDXP // 8) + gi, 0)),
        out_shape=jax.ShapeDtypeStruct((NTAB, DP), jnp.int32),
    )(temb, iemb, pemb)


@functools.partial(
    pl.kernel,
    out_type=jax.ShapeDtypeStruct((B * S, D), jnp.float32),
    mesh=plsc.VectorSubcoreMesh(
        core_axis_name="c", subcore_axis_name="s",
        num_cores=NC, num_subcores=NS),
    compiler_params=pltpu.CompilerParams(needs_layout_passes=False),
    scratch_types=[
        pltpu.VMEM((S, D), jnp.float32),
        pltpu.VMEM((S, D), jnp.float32),
        pltpu.VMEM((S, DP), jnp.int32),
        pltpu.VMEM((3 * MSTR,), jnp.int32),
        pltpu.VMEM((3 * D,), jnp.float32),
        pltpu.VMEM((MSTR + LANES,), jnp.int32),
        pltpu.SemaphoreType.DMA,
        pltpu.SemaphoreType.DMA,
        pltpu.SemaphoreType.DMA,
        pltpu.SemaphoreType.DMA,
    ],
)
def _sc_embed(gidx_hbm, styled_hbm, tab_hbm, out_hbm,
              blk0, blk1, blkI, gidxb, styb, pglist,
              gsem, osem0, osem1, fsem):
    wid = lax.axis_index("s") * NC + lax.axis_index("c")
    b0 = wid * NB
    iota = lax.iota(jnp.int32, LANES)
    himask = jnp.full((LANES,), -65536, jnp.int32)

    def meta_src(r):
        return (gidx_hbm.at[pl.ds((b0 + r) * S, S)],
                styled_hbm.at[pl.ds((b0 + r) * D, D)])

    def meta_dst(r):
        m = (r % 3)
        return (gidxb.at[pl.ds(m * MSTR, S)], styb.at[pl.ds(m * D, D)])

    def gather_pair(r):
        m = (r % 3) * MSTR
        return ((tab_hbm.at[gidxb.at[pl.ds(m, G0)]], blkI.at[pl.ds(0, G0)]),
                (tab_hbm.at[gidxb.at[pl.ds(m + G0, G1)]],
                 blkI.at[pl.ds(G0, G1)]))

    for r in (0, 1):
        for sx, dx in zip(meta_src(r), meta_dst(r)):
            pltpu.sync_copy(sx, dx)
    for sx, dx in gather_pair(0):
        pltpu.async_copy(sx, dx, gsem)

    def do_row(r, blk_p, osem_p):
        mb = (r % 3)

        @pl.when((r >= 1) & (r + 1 < NB))
        def _():
            for sx, dx in zip(meta_src(r + 1), meta_dst(r + 1)):
                pltpu.make_async_copy(sx, dx, fsem).wait()

        @pl.when(r + 2 < NB)
        def _():
            for sx, dx in zip(meta_src(r + 2), meta_dst(r + 2)):
                pltpu.async_copy(sx, dx, fsem)

        for sx, dx in gather_pair(r):
            pltpu.make_async_copy(sx, dx, gsem).wait()

        @pl.when(r >= 2)
        def _():
            pltpu.make_async_copy(
                blk_p, out_hbm.at[pl.ds((b0 + r - 2) * S, S)], osem_p).wait()

        def exp_body(tt, carry):
            for r8 in range(8):
                row = tt * 8 + r8
                for k in range(8):
                    v = blkI[row, pl.ds(k * LANES, LANES)]
                    lo = lax.bitcast_convert_type(v << 16, jnp.float32)
                    hi = lax.bitcast_convert_type(v & himask, jnp.float32)
                    blk_p[row, pl.ds(2 * k * LANES, LANES)] = lo
                    blk_p[row, pl.ds((2 * k + 1) * LANES, LANES)] = hi
            return carry

        lax.fori_loop(0, G0 // 8, exp_body, 0)

        @pl.when(r + 1 < NB)
        def _():
            sx, dx = gather_pair(r + 1)[0]
            pltpu.async_copy(sx, dx, gsem)

        lax.fori_loop(G0 // 8, S // 8, exp_body, 0)

        @pl.when(r + 1 < NB)
        def _():
            sx, dx = gather_pair(r + 1)[1]
            pltpu.async_copy(sx, dx, gsem)

        sty = tuple(styb[pl.ds(mb * D + k * LANES, LANES)]
                    for k in range(D // LANES))
        cnt = 0
        for w in range(S // LANES + 1):
            off = w * LANES
            gv = gidxb[pl.ds(mb * MSTR + off, LANES)]
            pm = (gv >= PAGE_LO) & (gv < PAGE_HI)
            if w == S // LANES:
                pm = pm & (iota < S - off)
            plsc.store_compressed(pglist.at[pl.ds(cnt, LANES)],
                                  iota + off, mask=pm)
            cnt = cnt + plsc.all_reduce_population_count(pm)[0]

        def fix_body(wi, carry):
            pg = pglist[pl.ds(wi * LANES, LANES)]
            for l in range(LANES):
                s_l = pg[l]

                @pl.when(wi * LANES + l < cnt)
                def _(_s=s_l):
                    for k in range(D // LANES):
                        blk_p[_s, pl.ds(k * LANES, LANES)] += sty[k]
            return carry

        lax.fori_loop(0, (cnt + LANES - 1) // LANES, fix_body, 0)

        pltpu.async_copy(blk_p, out_hbm.at[pl.ds((b0 + r) * S, S)], osem_p)

    def pair_body(h, carry):
        do_row(2 * h, blk0, osem0)
        do_row(2 * h + 1, blk1, osem1)
        return carry

    lax.fori_loop(0, NB // 2, pair_body, 0)

    for blk_p, osem_p, r in ((blk0, osem0, NB - 2), (blk1, osem1, NB - 1)):
        pltpu.make_async_copy(blk_p, out_hbm.at[pl.ds((b0 + r) * S, S)],
                              osem_p).wait()


def kernel(element_types, element_indices, style_vector, type_emb, idx_emb,
           W1, b1, W2, b2, pos_emb):
    types = element_types.astype(jnp.int32)
    inds = element_indices.astype(jnp.int32)
    styled, mask, gidx = _tc_pre(types, inds, style_vector, W1, b1, W2, b2)
    iemb_pad = jnp.pad(idx_emb, ((0, NIDXP - NIDX), (0, 0)))
    bigtab = _tc_bigtab(type_emb, iemb_pad, pos_emb)
    final = _sc_embed(gidx.reshape(-1), styled.reshape(-1), bigtab)
    return final.reshape(B, S, D), mask


# speedup vs baseline: 2.0791x; 2.0791x over previous
import functools

import jax
import jax.numpy as jnp
from jax import lax
from jax.experimental import pallas as pl
from jax.experimental.pallas import tpu as pltpu
from jax.experimental.pallas import tpu_sc as plsc

B, S, D = 4096, 200, 256
NTYPE, NIDX = 5, 50
NCOMBO = NTYPE * NIDX
NTAB = NCOMBO * S
NC, NS = 2, 16
NW = NC * NS
NB = B // NW
LANES = 16
G0, G1 = 104, 96
MSTR = 208
PAGE_LO, PAGE_HI = NIDX * S, 2 * NIDX * S


def _style_mask_body(types_ref, inds_ref, sv_ref, w1_ref, b1_ref, w2_ref,
                     b2_ref, styled_ref, mask_ref, gidx_ref):
    h = jnp.dot(sv_ref[...], w1_ref[...], preferred_element_type=jnp.float32)
    h = jnp.maximum(h + b1_ref[...][None, :], 0.0)
    styled = jnp.dot(h, w2_ref[...], preferred_element_type=jnp.float32)
    styled_ref[...] = styled + b2_ref[...][None, :]
    mask_ref[...] = types_ref[...] == 0
    s_iota = lax.broadcasted_iota(jnp.int32, (B, S), 1)
    gidx_ref[...] = (types_ref[...] * NIDX + inds_ref[...]) * S + s_iota


def _tc_pre(types, inds, style_vector, w1, b1, w2, b2):
    return pl.pallas_call(
        _style_mask_body,
        out_shape=[
            jax.ShapeDtypeStruct((B, D), jnp.float32),
            jax.ShapeDtypeStruct((B, S), jnp.bool_),
            jax.ShapeDtypeStruct((B, S), jnp.int32),
        ],
    )(types, inds, style_vector, w1, b1, w2, b2)


def _bigtab_body(temb_ref, iemb_ref, pemb_ref, out_ref):
    g = pl.program_id(0)
    trow = jnp.zeros((1, D), jnp.float32)
    for t in range(NTYPE):
        trow = trow + temb_ref[pl.ds(t, 1), :] * jnp.where(g == t, 1.0, 0.0)
    big = (iemb_ref[...][:, None, :] + pemb_ref[...][None, :, :]
           + trow[None, :, :])
    out_ref[...] = big.reshape(NIDX * S, D)


def _tc_bigtab(temb, iemb, pemb):
    return pl.pallas_call(
        _bigtab_body,
        grid=(NTYPE,),
        in_specs=[
            pl.BlockSpec((NTYPE, D), lambda g: (0, 0)),
            pl.BlockSpec((NIDX, D), lambda g: (0, 0)),
            pl.BlockSpec((S, D), lambda g: (0, 0)),
        ],
        out_specs=pl.BlockSpec((NIDX * S, D), lambda g: (g, 0)),
        out_shape=jax.ShapeDtypeStruct((NTAB, D), jnp.float32),
    )(temb, iemb, pemb)


@functools.partial(
    pl.kernel,
    out_type=jax.ShapeDtypeStruct((B * S, D), jnp.float32),
    mesh=plsc.VectorSubcoreMesh(
        core_axis_name="c", subcore_axis_name="s",
        num_cores=NC, num_subcores=NS),
    compiler_params=pltpu.CompilerParams(needs_layout_passes=False),
    scratch_types=[
        pltpu.VMEM((S, D), jnp.float32),
        pltpu.VMEM((S, D), jnp.float32),
        pltpu.VMEM((3 * MSTR,), jnp.int32),
        pltpu.VMEM((3 * D,), jnp.float32),
        pltpu.VMEM((MSTR + LANES,), jnp.int32),
        pltpu.SemaphoreType.DMA,
        pltpu.SemaphoreType.DMA,
        pltpu.SemaphoreType.DMA,
        pltpu.SemaphoreType.DMA,
        pltpu.SemaphoreType.DMA,
    ],
)
def _sc_embed(gidx_hbm, styled_hbm, tab_hbm, out_hbm,
              blk0, blk1, gidxb, styb, pglist,
              gsem0, gsem1, osem0, osem1, fsem):
    wid = lax.axis_index("s") * NC + lax.axis_index("c")
    b0 = wid * NB
    iota = lax.iota(jnp.int32, LANES)

    def meta_src(r):
        return (gidx_hbm.at[pl.ds((b0 + r) * S, S)],
                styled_hbm.at[pl.ds((b0 + r) * D, D)])

    def meta_dst(r):
        m = (r % 3)
        return (gidxb.at[pl.ds(m * MSTR, S)], styb.at[pl.ds(m * D, D)])

    def gather_pair(r, blk):
        m = (r % 3) * MSTR
        return ((tab_hbm.at[gidxb.at[pl.ds(m, G0)]], blk.at[pl.ds(0, G0)]),
                (tab_hbm.at[gidxb.at[pl.ds(m + G0, G1)]],
                 blk.at[pl.ds(G0, G1)]))

    for r in (0, 1):
        for sx, dx in zip(meta_src(r), meta_dst(r)):
            pltpu.sync_copy(sx, dx)
    for sx, dx in gather_pair(0, blk0):
        pltpu.async_copy(sx, dx, gsem0)

    def do_row(r, p, blk_p, blk_o, gsem_p, gsem_o, osem_p, osem_o):
        mb = (r % 3)

        @pl.when((r >= 1) & (r + 1 < NB))
        def _():
            for sx, dx in zip(meta_src(r + 1), meta_dst(r + 1)):
                pltpu.make_async_copy(sx, dx, fsem).wait()

        @pl.when(r + 2 < NB)
        def _():
            for sx, dx in zip(meta_src(r + 2), meta_dst(r + 2)):
                pltpu.async_copy(sx, dx, fsem)

        @pl.when(r >= 1)
        def _():
            pltpu.make_async_copy(
                blk_o, out_hbm.at[pl.ds((b0 + r - 1) * S, S)], osem_o).wait()

        @pl.when(r + 1 < NB)
        def _():
            for sx, dx in gather_pair(r + 1, blk_o):
                pltpu.async_copy(sx, dx, gsem_o)

        for sx, dx in gather_pair(r, blk_p):
            pltpu.make_async_copy(sx, dx, gsem_p).wait()

        sty = tuple(styb[pl.ds(mb * D + k * LANES, LANES)]
                    for k in range(D // LANES))
        cnt = 0
        for w in range(S // LANES + 1):
            off = w * LANES
            gv = gidxb[pl.ds(mb * MSTR + off, LANES)]
            pm = (gv >= PAGE_LO) & (gv < PAGE_HI)
            if w == S // LANES:
                pm = pm & (iota < S - off)
            plsc.store_compressed(pglist.at[pl.ds(cnt, LANES)],
                                  iota + off, mask=pm)
            cnt = cnt + plsc.all_reduce_population_count(pm)[0]

        def fix_body(wi, carry):
            pg = pglist[pl.ds(wi * LANES, LANES)]
            for l in range(LANES):
                s_l = pg[l]

                @pl.when(wi * LANES + l < cnt)
                def _(_s=s_l):
                    for k in range(D // LANES):
                        blk_p[_s, pl.ds(k * LANES, LANES)] += sty[k]
            return carry

        lax.fori_loop(0, (cnt + LANES - 1) // LANES, fix_body, 0)

        pltpu.async_copy(blk_p, out_hbm.at[pl.ds((b0 + r) * S, S)], osem_p)

    def pair_body(h, carry):
        do_row(2 * h, 0, blk0, blk1, gsem0, gsem1, osem0, osem1)
        do_row(2 * h + 1, 1, blk1, blk0, gsem1, gsem0, osem1, osem0)
        return carry

    lax.fori_loop(0, NB // 2, pair_body, 0)

    pltpu.make_async_copy(blk1, out_hbm.at[pl.ds((b0 + NB - 1) * S, S)],
                          osem1).wait()


def kernel(element_types, element_indices, style_vector, type_emb, idx_emb,
           W1, b1, W2, b2, pos_emb):
    types = element_types.astype(jnp.int32)
    inds = element_indices.astype(jnp.int32)
    styled, mask, gidx = _tc_pre(types, inds, style_vector, W1, b1, W2, b2)
    bigtab = _tc_bigtab(type_emb, idx_emb, pos_emb)
    final = _sc_embed(gidx.reshape(-1), styled.reshape(-1), bigtab)
    return final.reshape(B, S, D), mask
